# baseline (speedup 1.0000x reference)
import functools

import jax
import jax.numpy as jnp
import numpy as np
from jax.experimental import pallas as pl
from jax.experimental.pallas import tpu as pltpu

N = 4096
NFEAT = 256
NHID = 256
OUT_DIM = 128
BN_EPS = 1e-5

K_BLK = 256
N_K = N // K_BLK
GRP = 4
PAN = 1024
N_PAN = N // PAN


def _fused_body(a_ref, xtb_ref, w1a_ref, c1a_ref, w2a_ref, c2a_ref,
                w1b_ref, c1b_ref, w2b_ref, c2b_ref, out_ref,
                mask_ref, acc_ref, htb_ref):
    k = pl.program_id(0)
    m = (a_ref[...] > 0.0).astype(jnp.bfloat16)
    mask_ref[pl.ds(k * K_BLK, K_BLK), :] = m
    @pl.when((k % GRP == GRP - 1) & (k != N_K - 1))
    def _():
        k0 = (k - (GRP - 1)) * K_BLK
        part = jnp.dot(
            xtb_ref[:, pl.ds(k0, GRP * K_BLK)],
            mask_ref[pl.ds(k0, GRP * K_BLK), :],
            preferred_element_type=jnp.float32)

        @pl.when(k == GRP - 1)
        def _():
            acc_ref[...] = part

        @pl.when(k != GRP - 1)
        def _():
            acc_ref[...] += part

    @pl.when(k == N_K - 1)
    def _():
        k0 = (k - (GRP - 1)) * K_BLK
        last = jnp.dot(
            xtb_ref[:, pl.ds(k0, GRP * K_BLK)],
            mask_ref[pl.ds(k0, GRP * K_BLK), :],
            preferred_element_type=jnp.float32)
        acc_ref[...] += last + xtb_ref[...].astype(jnp.float32)
        for p in range(N_PAN):
            sl = slice(p * PAN, (p + 1) * PAN)
            h = jnp.dot(w1a_ref[...], acc_ref[:, sl].astype(jnp.bfloat16),
                        preferred_element_type=jnp.float32) + c1a_ref[...]
            h = jnp.maximum(h, 0.0)
            o = jnp.dot(w2a_ref[...], h.astype(jnp.bfloat16),
                        preferred_element_type=jnp.float32) + c2a_ref[...]
            htb_ref[:, sl] = jnp.maximum(o, 0.0).astype(jnp.bfloat16)
        htb = htb_ref[...]
        for p in range(N_PAN):
            sl = slice(p * PAN, (p + 1) * PAN)
            aggr2 = jnp.dot(htb, mask_ref[:, sl],
                            preferred_element_type=jnp.float32)
            aggr2 = aggr2 + htb[:, sl].astype(jnp.float32)
            h2 = jnp.dot(w1b_ref[...], aggr2.astype(jnp.bfloat16),
                         preferred_element_type=jnp.float32) + c1b_ref[...]
            h2 = jnp.maximum(h2, 0.0)
            out_ref[:, sl] = jnp.dot(
                w2b_ref[...], h2.astype(jnp.bfloat16),
                preferred_element_type=jnp.float32) + c2b_ref[...]


def kernel(x, A, W1a, b1a, g1a, be1a, W2a, b2a, W1b, b1b, g1b, be1b, W2b, b2b):
    inv = np.float32(1.0 / np.sqrt(1.0 + BN_EPS))
    gs_a = g1a * inv
    w1a = (W1a * gs_a[None, :]).T.astype(jnp.bfloat16)
    c1a = (b1a * gs_a + be1a)[:, None]
    gs_b = g1b * inv
    w1b = (W1b * gs_b[None, :]).T.astype(jnp.bfloat16)
    c1b = (b1b * gs_b + be1b)[:, None]
    w2a = W2a.T.astype(jnp.bfloat16)
    w2b = W2b.T.astype(jnp.bfloat16)
    c2a = b2a[:, None]
    c2b = b2b[:, None]

    xTb = x.T.astype(jnp.bfloat16)

    full = lambda shape: pl.BlockSpec(shape, lambda k: (0, 0))
    outT = pl.pallas_call(
        _fused_body,
        grid=(N_K,),
        in_specs=[
            pl.BlockSpec((K_BLK, N), lambda k: (k, 0)),
            full((NFEAT, N)),
            full(w1a.shape), full(c1a.shape),
            full(w2a.shape), full(c2a.shape),
            full(w1b.shape), full(c1b.shape),
            full(w2b.shape), full(c2b.shape),
        ],
        out_specs=full((OUT_DIM, N)),
        out_shape=jax.ShapeDtypeStruct((OUT_DIM, N), jnp.float32),
        scratch_shapes=[
            pltpu.VMEM((N, N), jnp.bfloat16),
            pltpu.VMEM((NFEAT, N), jnp.float32),
            pltpu.VMEM((NHID, N), jnp.bfloat16),
        ],
    )(A, xTb, w1a, c1a, w2a, c2a, w1b, c1b, w2b, c2b)
    return outT.T


# speedup vs baseline: 1.0007x; 1.0007x over previous
import functools

import jax
import jax.numpy as jnp
import numpy as np
from jax.experimental import pallas as pl
from jax.experimental.pallas import tpu as pltpu

N = 4096
NFEAT = 256
NHID = 256
OUT_DIM = 128
BN_EPS = 1e-5

K_BLK = 256
N_K = N // K_BLK
PAN = 512
N_PAN = N // PAN


def _fused_body(a_ref, xtb_ref, w1a_ref, c1a_ref, w2a_ref, c2a_ref,
                w1b_ref, c1b_ref, w2b_ref, c2b_ref, out_ref,
                mask_ref, acc_ref, htb_ref):
    k = pl.program_id(0)
    m = (a_ref[...] > 0.0).astype(jnp.bfloat16)
    mask_ref[pl.ds(k * K_BLK, K_BLK), :] = m
    part = jnp.dot(xtb_ref[:, pl.ds(k * K_BLK, K_BLK)], m,
                   preferred_element_type=jnp.float32)

    @pl.when(k == 0)
    def _():
        acc_ref[...] = part

    @pl.when((k != 0) & (k != N_K - 1))
    def _():
        acc_ref[...] += part

    @pl.when(k == N_K - 1)
    def _():
        acc_ref[...] += part + xtb_ref[...].astype(jnp.float32)
        for p in range(N_PAN):
            sl = slice(p * PAN, (p + 1) * PAN)
            h = jnp.dot(w1a_ref[...], acc_ref[:, sl].astype(jnp.bfloat16),
                        preferred_element_type=jnp.float32) + c1a_ref[...]
            h = jnp.maximum(h, 0.0)
            o = jnp.dot(w2a_ref[...], h.astype(jnp.bfloat16),
                        preferred_element_type=jnp.float32) + c2a_ref[...]
            htb_ref[:, sl] = jnp.maximum(o, 0.0).astype(jnp.bfloat16)
        htb = htb_ref[...]
        for p in range(N_PAN):
            sl = slice(p * PAN, (p + 1) * PAN)
            aggr2 = jnp.dot(htb, mask_ref[:, sl],
                            preferred_element_type=jnp.float32)
            aggr2 = aggr2 + htb[:, sl].astype(jnp.float32)
            h2 = jnp.dot(w1b_ref[...], aggr2.astype(jnp.bfloat16),
                         preferred_element_type=jnp.float32) + c1b_ref[...]
            h2 = jnp.maximum(h2, 0.0)
            out_ref[:, sl] = jnp.dot(
                w2b_ref[...], h2.astype(jnp.bfloat16),
                preferred_element_type=jnp.float32) + c2b_ref[...]


def kernel(x, A, W1a, b1a, g1a, be1a, W2a, b2a, W1b, b1b, g1b, be1b, W2b, b2b):
    inv = np.float32(1.0 / np.sqrt(1.0 + BN_EPS))
    gs_a = g1a * inv
    w1a = (W1a * gs_a[None, :]).T.astype(jnp.bfloat16)
    c1a = (b1a * gs_a + be1a)[:, None]
    gs_b = g1b * inv
    w1b = (W1b * gs_b[None, :]).T.astype(jnp.bfloat16)
    c1b = (b1b * gs_b + be1b)[:, None]
    w2a = W2a.T.astype(jnp.bfloat16)
    w2b = W2b.T.astype(jnp.bfloat16)
    c2a = b2a[:, None]
    c2b = b2b[:, None]

    xTb = x.T.astype(jnp.bfloat16)

    full = lambda shape: pl.BlockSpec(shape, lambda k: (0, 0))
    outT = pl.pallas_call(
        _fused_body,
        grid=(N_K,),
        in_specs=[
            pl.BlockSpec((K_BLK, N), lambda k: (k, 0)),
            full((NFEAT, N)),
            full(w1a.shape), full(c1a.shape),
            full(w2a.shape), full(c2a.shape),
            full(w1b.shape), full(c1b.shape),
            full(w2b.shape), full(c2b.shape),
        ],
        out_specs=full((OUT_DIM, N)),
        out_shape=jax.ShapeDtypeStruct((OUT_DIM, N), jnp.float32),
        scratch_shapes=[
            pltpu.VMEM((N, N), jnp.bfloat16),
            pltpu.VMEM((NFEAT, N), jnp.float32),
            pltpu.VMEM((NHID, N), jnp.bfloat16),
        ],
    )(A, xTb, w1a, c1a, w2a, c2a, w1b, c1b, w2b, c2b)
    return outT.T


# speedup vs baseline: 1.0441x; 1.0433x over previous
import functools

import jax
import jax.numpy as jnp
import numpy as np
from jax.experimental import pallas as pl
from jax.experimental.pallas import tpu as pltpu

N = 4096
NFEAT = 256
NHID = 256
OUT_DIM = 128
BN_EPS = 1e-5

K_BLK = 256
N_K = N // K_BLK
PAN = 1024
N_PAN = N // PAN


def _fused_body(a_ref, xtb_ref, w1a_ref, c1a_ref, w2a_ref, c2a_ref,
                w1b_ref, c1b_ref, w2b_ref, c2b_ref, out_ref,
                mask_ref, acc_ref, htb_ref):
    k = pl.program_id(0)
    m = (a_ref[...] > 0.0).astype(jnp.bfloat16)
    mask_ref[pl.ds(k * K_BLK, K_BLK), :] = m
    part = jnp.dot(xtb_ref[:, pl.ds(k * K_BLK, K_BLK)], m,
                   preferred_element_type=jnp.float32)

    @pl.when(k == 0)
    def _():
        acc_ref[...] = part

    @pl.when((k != 0) & (k != N_K - 1))
    def _():
        acc_ref[...] += part

    @pl.when(k == N_K - 1)
    def _():
        acc_ref[...] += part + xtb_ref[...].astype(jnp.float32)
        for p in range(N_PAN):
            sl = slice(p * PAN, (p + 1) * PAN)
            h = jnp.dot(w1a_ref[...], acc_ref[:, sl].astype(jnp.bfloat16),
                        preferred_element_type=jnp.float32) + c1a_ref[...]
            h = jnp.maximum(h, 0.0)
            o = jnp.dot(w2a_ref[...], h.astype(jnp.bfloat16),
                        preferred_element_type=jnp.float32) + c2a_ref[...]
            htb_ref[:, sl] = jnp.maximum(o, 0.0).astype(jnp.bfloat16)
        htb = htb_ref[...]
        for p in range(N_PAN):
            sl = slice(p * PAN, (p + 1) * PAN)
            aggr2 = jnp.dot(htb, mask_ref[:, sl],
                            preferred_element_type=jnp.float32)
            aggr2 = aggr2 + htb[:, sl].astype(jnp.float32)
            h2 = jnp.dot(w1b_ref[...], aggr2.astype(jnp.bfloat16),
                         preferred_element_type=jnp.float32) + c1b_ref[...]
            h2 = jnp.maximum(h2, 0.0)
            out_ref[:, sl] = jnp.dot(
                w2b_ref[...], h2.astype(jnp.bfloat16),
                preferred_element_type=jnp.float32) + c2b_ref[...]


def kernel(x, A, W1a, b1a, g1a, be1a, W2a, b2a, W1b, b1b, g1b, be1b, W2b, b2b):
    inv = np.float32(1.0 / np.sqrt(1.0 + BN_EPS))
    gs_a = g1a * inv
    w1a = (W1a * gs_a[None, :]).T.astype(jnp.bfloat16)
    c1a = (b1a * gs_a + be1a)[:, None]
    gs_b = g1b * inv
    w1b = (W1b * gs_b[None, :]).T.astype(jnp.bfloat16)
    c1b = (b1b * gs_b + be1b)[:, None]
    w2a = W2a.T.astype(jnp.bfloat16)
    w2b = W2b.T.astype(jnp.bfloat16)
    c2a = b2a[:, None]
    c2b = b2b[:, None]

    xTb = x.T.astype(jnp.bfloat16)

    full = lambda shape: pl.BlockSpec(shape, lambda k: (0, 0))
    outT = pl.pallas_call(
        _fused_body,
        grid=(N_K,),
        in_specs=[
            pl.BlockSpec((K_BLK, N), lambda k: (k, 0)),
            full((NFEAT, N)),
            full(w1a.shape), full(c1a.shape),
            full(w2a.shape), full(c2a.shape),
            full(w1b.shape), full(c1b.shape),
            full(w2b.shape), full(c2b.shape),
        ],
        out_specs=full((OUT_DIM, N)),
        out_shape=jax.ShapeDtypeStruct((OUT_DIM, N), jnp.float32),
        scratch_shapes=[
            pltpu.VMEM((N, N), jnp.bfloat16),
            pltpu.VMEM((NFEAT, N), jnp.float32),
            pltpu.VMEM((NHID, N), jnp.bfloat16),
        ],
    )(A, xTb, w1a, c1a, w2a, c2a, w1b, c1b, w2b, c2b)
    return outT.T
